# baseline (speedup 1.0000x reference)
import functools

import jax
import jax.numpy as jnp
from jax import lax
from jax.experimental import pallas as pl
from jax.experimental.pallas import tpu as pltpu
from jax.experimental.pallas import tpu_sc as plsc

BATCH = 16384
D_FEAT = 128
N_CLASSES = 1000

NUM_CORES = 2
NUM_SUBCORES = 16
NW = NUM_CORES * NUM_SUBCORES
BPW = BATCH // NW
CHUNK = 128
NCHUNK = BPW // CHUNK

_sc_mesh = plsc.VectorSubcoreMesh(core_axis_name="c", subcore_axis_name="s")


@functools.partial(
    pl.kernel,
    mesh=_sc_mesh,
    out_type=jax.ShapeDtypeStruct((BATCH, D_FEAT), jnp.float32),
    scratch_types=[
        pltpu.VMEM((NCHUNK, CHUNK), jnp.int32),
        pltpu.VMEM((BPW, D_FEAT), jnp.float32),
        pltpu.SemaphoreType.DMA,
    ],
)
def _sc_gather(idx_hbm, table_hbm, out_hbm, idx_v, rows_v, sem):
    wid = lax.axis_index("s") * NUM_CORES + lax.axis_index("c")
    base = wid * BPW
    pltpu.sync_copy(idx_hbm.at[wid], idx_v)
    copies = []
    for j in range(NCHUNK):
        copies.append(
            pltpu.async_copy(
                table_hbm.at[idx_v.at[j]],
                rows_v.at[pl.ds(j * CHUNK, CHUNK)],
                sem,
            )
        )
    for c in copies:
        c.wait()
    pltpu.sync_copy(rows_v, out_hbm.at[pl.ds(base, BPW)])


_BM = 2048
_NSPLIT = 4
_ROWS = _BM // _NSPLIT


_BN = 1024


def _tc_matmul_body(w_ref, c_ref, b_ref, o_ref):
    o_ref[...] = (
        lax.dot_general(
            w_ref[...],
            c_ref[...],
            (((1,), (1,)), ((), ())),
            preferred_element_type=jnp.float32,
        )
        + b_ref[...]
    )


def _tc_linear_t(W, concepts, bcol):
    return pl.pallas_call(
        _tc_matmul_body,
        grid=(BATCH // _BN,),
        in_specs=[
            pl.BlockSpec((N_CLASSES, D_FEAT), lambda i: (0, 0)),
            pl.BlockSpec((_BN, D_FEAT), lambda i: (i, 0)),
            pl.BlockSpec((N_CLASSES, 1), lambda i: (0, 0)),
        ],
        out_specs=pl.BlockSpec((N_CLASSES, _BN), lambda i: (0, i)),
        out_shape=jax.ShapeDtypeStruct((N_CLASSES, BATCH), jnp.float32),
    )(W, concepts, bcol)


@jax.jit
def kernel(x, clip_features, W, b):
    idx = x.astype(jnp.int32).reshape(NW, NCHUNK, CHUNK)
    concepts = _sc_gather(idx, clip_features)
    preds_t = _tc_linear_t(W, concepts, b.reshape(N_CLASSES, 1))
    return concepts, concepts, preds_t.T


# speedup vs baseline: 1.0656x; 1.0656x over previous
import functools

import jax
import jax.numpy as jnp
from jax import lax
from jax.experimental import pallas as pl
from jax.experimental.pallas import tpu as pltpu
from jax.experimental.pallas import tpu_sc as plsc

BATCH = 16384
D_FEAT = 128
N_CLASSES = 1000

NUM_CORES = 2
NUM_SUBCORES = 16
NW = NUM_CORES * NUM_SUBCORES
BPW = BATCH // NW
CHUNK = 128
NCHUNK = BPW // CHUNK

_sc_mesh = plsc.VectorSubcoreMesh(core_axis_name="c", subcore_axis_name="s")


@functools.partial(
    pl.kernel,
    mesh=_sc_mesh,
    out_type=jax.ShapeDtypeStruct((BATCH, D_FEAT), jnp.float32),
    scratch_types=[
        pltpu.VMEM((NCHUNK, CHUNK), jnp.int32),
        pltpu.VMEM((BPW, D_FEAT), jnp.float32),
        pltpu.SemaphoreType.DMA,
    ],
)
def _sc_gather(idx_hbm, table_hbm, out_hbm, idx_v, rows_v, sem):
    wid = lax.axis_index("s") * NUM_CORES + lax.axis_index("c")
    base = wid * BPW
    pltpu.sync_copy(idx_hbm.at[wid], idx_v)
    copies = []
    for j in range(NCHUNK):
        copies.append(
            pltpu.async_copy(
                table_hbm.at[idx_v.at[j]],
                rows_v.at[pl.ds(j * CHUNK, CHUNK)],
                sem,
            )
        )
    for c in copies:
        c.wait()
    pltpu.sync_copy(rows_v, out_hbm.at[pl.ds(base, BPW)])


_BM = 2048
_NSPLIT = 4
_ROWS = _BM // _NSPLIT


_BN = 2048


def _tc_matmul_body(w_ref, c_ref, b_ref, o_ref):
    o_ref[...] = (
        lax.dot_general(
            w_ref[...],
            c_ref[...],
            (((1,), (1,)), ((), ())),
            preferred_element_type=jnp.float32,
        )
        + b_ref[...]
    )


def _tc_linear_t(W, concepts, bcol):
    return pl.pallas_call(
        _tc_matmul_body,
        grid=(BATCH // _BN,),
        in_specs=[
            pl.BlockSpec((N_CLASSES, D_FEAT), lambda i: (0, 0)),
            pl.BlockSpec((_BN, D_FEAT), lambda i: (i, 0)),
            pl.BlockSpec((N_CLASSES, 1), lambda i: (0, 0)),
        ],
        out_specs=pl.BlockSpec((N_CLASSES, _BN), lambda i: (0, i)),
        out_shape=jax.ShapeDtypeStruct((N_CLASSES, BATCH), jnp.float32),
    )(W, concepts, bcol)


@jax.jit
def kernel(x, clip_features, W, b):
    idx = x.astype(jnp.int32).reshape(NW, NCHUNK, CHUNK)
    concepts = _sc_gather(idx, clip_features)
    preds_t = _tc_linear_t(W, concepts, b.reshape(N_CLASSES, 1))
    return concepts, concepts, preds_t.T
